# baseline (speedup 1.0000x reference)
import functools

import jax
import jax.numpy as jnp
from jax import lax
from jax.experimental import pallas as pl
from jax.experimental.pallas import tpu as pltpu
from jax.experimental.pallas import tpu_sc as plsc

NC = 2
NS = 16
NW = NC * NS
GW = 128
D = 64
NB = 4


def _build(ch):
  mesh = plsc.VectorSubcoreMesh(core_axis_name="c", subcore_axis_name="s")

  @functools.partial(
      pl.kernel,
      out_type=jax.ShapeDtypeStruct((NW, ch, GW, D), jnp.float32),
      mesh=mesh,
      compiler_params=pltpu.CompilerParams(use_tc_tiling_on_sc=False),
      scratch_types=[
          pltpu.VMEM((ch, GW), jnp.int32),
          pltpu.VMEM((NB, GW, D), jnp.float32),
      ] + [pltpu.SemaphoreType.DMA] * (2 * NB),
  )
  def k(idx_hbm, table_hbm, out_hbm, idx_v, rows_v, *sems):
    gsem = sems[:NB]
    wsem = sems[NB:]
    wid = lax.axis_index("s") * NC + lax.axis_index("c")
    my_out = out_hbm.at[wid]
    pltpu.sync_copy(idx_hbm.at[wid], idx_v)

    for b in range(NB):
      pltpu.async_copy(table_hbm.at[idx_v.at[b]], rows_v.at[b], gsem[b])

    def outer(t, carry):
      j0 = t * NB
      for b in range(NB):
        pltpu.make_async_copy(
            table_hbm.at[idx_v.at[0]], rows_v.at[b], gsem[b]).wait()
        pltpu.async_copy(rows_v.at[b], my_out.at[j0 + b], wsem[b])
      for b in range(NB):
        jn = j0 + NB + b

        @pl.when(jn < ch)
        def _start_next(b=b, jn=jn):
          pltpu.make_async_copy(rows_v.at[b], my_out.at[0], wsem[b]).wait()
          pltpu.async_copy(table_hbm.at[idx_v.at[jn]], rows_v.at[b], gsem[b])

      return carry

    lax.fori_loop(0, ch // NB, outer, 0)

    for b in range(NB):
      pltpu.make_async_copy(rows_v.at[b], my_out.at[0], wsem[b]).wait()

  return k


def kernel(indices, table):
  nb, ns = indices.shape
  total = nb * ns
  ch = total // (NW * GW)
  idx3 = indices.reshape(NW, ch, GW)
  out = _build(ch)(idx3, table)
  return out.reshape(nb, ns, D)


# speedup vs baseline: 1.0004x; 1.0004x over previous
import functools

import jax
import jax.numpy as jnp
from jax import lax
from jax.experimental import pallas as pl
from jax.experimental.pallas import tpu as pltpu
from jax.experimental.pallas import tpu_sc as plsc

NC = 2
NS = 16
NW = NC * NS
GW = 128
D = 64
SG = 4
NSLOT = 2


def _build(cb):
  mesh = plsc.VectorSubcoreMesh(core_axis_name="c", subcore_axis_name="s")

  @functools.partial(
      pl.kernel,
      out_type=jax.ShapeDtypeStruct((NW, cb, SG, GW, D), jnp.float32),
      mesh=mesh,
      compiler_params=pltpu.CompilerParams(use_tc_tiling_on_sc=False),
      scratch_types=[
          pltpu.VMEM((cb * SG, GW), jnp.int32),
          pltpu.VMEM((NSLOT, SG, GW, D), jnp.float32),
      ] + [pltpu.SemaphoreType.DMA] * (2 * NSLOT),
  )
  def k(idx_hbm, table_hbm, out_hbm, idx_v, rows_v, *sems):
    gsem = sems[:NSLOT]
    wsem = sems[NSLOT:]
    wid = lax.axis_index("s") * NC + lax.axis_index("c")
    my_out = out_hbm.at[wid]
    pltpu.sync_copy(idx_hbm.at[wid], idx_v)

    def fire(c, s):
      for g in range(SG):
        pltpu.async_copy(
            table_hbm.at[idx_v.at[c * SG + g]], rows_v.at[s, g], gsem[s])

    for s in range(NSLOT):
      fire(s, s)

    def outer(t, carry):
      for s in range(NSLOT):
        c = t * NSLOT + s
        pltpu.make_async_copy(my_out.at[0], rows_v.at[s], gsem[s]).wait()
        pltpu.async_copy(rows_v.at[s], my_out.at[c], wsem[s])

        @pl.when(c + NSLOT < cb)
        def _start_next(c=c, s=s):
          pltpu.make_async_copy(rows_v.at[s], my_out.at[0], wsem[s]).wait()
          fire(c + NSLOT, s)

      return carry

    lax.fori_loop(0, cb // NSLOT, outer, 0)

    for s in range(NSLOT):
      pltpu.make_async_copy(rows_v.at[s], my_out.at[0], wsem[s]).wait()

  return k


def kernel(indices, table):
  nb, ns = indices.shape
  total = nb * ns
  cb = total // (NW * SG * GW)
  idx3 = indices.reshape(NW, cb * SG, GW)
  out = _build(cb)(idx3, table)
  return out.reshape(nb, ns, D)


# speedup vs baseline: 1.0016x; 1.0012x over previous
import functools

import jax
import jax.numpy as jnp
from jax import lax
from jax.experimental import pallas as pl
from jax.experimental.pallas import tpu as pltpu
from jax.experimental.pallas import tpu_sc as plsc

NC = 2
NS = 16
NW = NC * NS
GW = 128
NSLOT = 4


def _build(nrows, seq, d):
  rpw = nrows // NW
  cuts = list(range(0, seq, GW))
  mesh = plsc.VectorSubcoreMesh(core_axis_name="c", subcore_axis_name="s")

  @functools.partial(
      pl.kernel,
      out_type=jax.ShapeDtypeStruct((nrows, seq, d), jnp.float32),
      mesh=mesh,
      compiler_params=pltpu.CompilerParams(use_tc_tiling_on_sc=False),
      scratch_types=[
          pltpu.VMEM((rpw, seq), jnp.int32),
          pltpu.VMEM((NSLOT, seq, d), jnp.float32),
      ] + [pltpu.SemaphoreType.DMA] * (2 * NSLOT),
  )
  def k(idx_hbm, table_hbm, out_hbm, idx_v, rows_v, *sems):
    gsem = sems[:NSLOT]
    wsem = sems[NSLOT:]
    wid = lax.axis_index("s") * NC + lax.axis_index("c")
    row0 = wid * rpw
    pltpu.sync_copy(idx_hbm.at[pl.ds(row0, rpw)], idx_v)

    def fire(r, s):
      for c in cuts:
        n = min(GW, seq - c)
        pltpu.async_copy(
            table_hbm.at[idx_v.at[r, pl.ds(c, n)]],
            rows_v.at[s, pl.ds(c, n)], gsem[s])

    for s in range(NSLOT):
      fire(s, s)

    def outer(t, carry):
      for s in range(NSLOT):
        r = t * NSLOT + s
        pltpu.make_async_copy(out_hbm.at[0], rows_v.at[s], gsem[s]).wait()
        pltpu.async_copy(rows_v.at[s], out_hbm.at[row0 + r], wsem[s])

        @pl.when(r + NSLOT < rpw)
        def _start_next(r=r, s=s):
          pltpu.make_async_copy(rows_v.at[s], out_hbm.at[0], wsem[s]).wait()
          fire(r + NSLOT, s)

      return carry

    lax.fori_loop(0, rpw // NSLOT, outer, 0)

    for s in range(NSLOT):
      pltpu.make_async_copy(rows_v.at[s], out_hbm.at[0], wsem[s]).wait()

  return k


def kernel(indices, table):
  nrows, seq = indices.shape
  d = table.shape[1]
  return _build(nrows, seq, d)(indices, table)


# speedup vs baseline: 1.0019x; 1.0003x over previous
import functools

import jax
import jax.numpy as jnp
from jax import lax
from jax.experimental import pallas as pl
from jax.experimental.pallas import tpu as pltpu
from jax.experimental.pallas import tpu_sc as plsc

NC = 2
NS = 16
NW = NC * NS
G = 128
CH = 512
NSLOT = 2


def _build(n, d):
  ipw = n // NW
  nch = ipw // CH
  mesh = plsc.VectorSubcoreMesh(core_axis_name="c", subcore_axis_name="s")

  @functools.partial(
      pl.kernel,
      out_type=jax.ShapeDtypeStruct((n, d), jnp.float32),
      mesh=mesh,
      compiler_params=pltpu.CompilerParams(use_tc_tiling_on_sc=False),
      scratch_types=[
          pltpu.VMEM((ipw,), jnp.int32),
          pltpu.VMEM((NSLOT, CH, d), jnp.float32),
      ] + [pltpu.SemaphoreType.DMA] * (2 * NSLOT),
  )
  def k(idx_hbm, table_hbm, out_hbm, idx_v, rows_v, *sems):
    gsem = sems[:NSLOT]
    wsem = sems[NSLOT:]
    wid = lax.axis_index("s") * NC + lax.axis_index("c")
    base = wid * ipw
    pltpu.sync_copy(idx_hbm.at[pl.ds(base, ipw)], idx_v)

    def fire(c, s):
      for j in range(CH // G):
        off = c * CH + j * G
        pltpu.async_copy(
            table_hbm.at[idx_v.at[pl.ds(off, G)]],
            rows_v.at[s, pl.ds(j * G, G)], gsem[s])

    for s in range(NSLOT):
      fire(s, s)

    def outer(t, carry):
      for s in range(NSLOT):
        c = t * NSLOT + s
        pltpu.make_async_copy(
            out_hbm.at[pl.ds(0, CH)], rows_v.at[s], gsem[s]).wait()
        pltpu.async_copy(
            rows_v.at[s], out_hbm.at[pl.ds(base + c * CH, CH)], wsem[s])

        @pl.when(c + NSLOT < nch)
        def _start_next(c=c, s=s):
          pltpu.make_async_copy(
              rows_v.at[s], out_hbm.at[pl.ds(0, CH)], wsem[s]).wait()
          fire(c + NSLOT, s)

      return carry

    lax.fori_loop(0, nch // NSLOT, outer, 0)

    for s in range(NSLOT):
      pltpu.make_async_copy(
          rows_v.at[s], out_hbm.at[pl.ds(0, CH)], wsem[s]).wait()

  return k


def kernel(indices, table):
  n = indices.size
  d = table.shape[1]
  out = _build(n, d)(indices.reshape(n), table)
  return out.reshape(*indices.shape, d)


# speedup vs baseline: 1.0031x; 1.0012x over previous
import functools

import jax
import jax.numpy as jnp
from jax import lax
from jax.experimental import pallas as pl
from jax.experimental.pallas import tpu as pltpu
from jax.experimental.pallas import tpu_sc as plsc

NC = 2
NS = 16
NW = NC * NS
G = 128
CH = 256
NSLOT = 5


def _build(n, d):
  ipw = n // NW
  nch = ipw // CH
  mesh = plsc.VectorSubcoreMesh(core_axis_name="c", subcore_axis_name="s")

  @functools.partial(
      pl.kernel,
      out_type=jax.ShapeDtypeStruct((n, d), jnp.float32),
      mesh=mesh,
      compiler_params=pltpu.CompilerParams(use_tc_tiling_on_sc=False),
      scratch_types=[
          pltpu.VMEM((ipw,), jnp.int32),
          pltpu.VMEM((NSLOT, CH, d), jnp.float32),
      ] + [pltpu.SemaphoreType.DMA] * (2 * NSLOT),
  )
  def k(idx_hbm, table_hbm, out_hbm, idx_v, rows_v, *sems):
    gsem = sems[:NSLOT]
    wsem = sems[NSLOT:]
    wid = lax.axis_index("s") * NC + lax.axis_index("c")
    base = wid * ipw
    pltpu.sync_copy(idx_hbm.at[pl.ds(base, ipw)], idx_v)

    def fire(c, s):
      for j in range(CH // G):
        pltpu.async_copy(
            table_hbm.at[idx_v.at[pl.ds(c * CH + j * G, G)]],
            rows_v.at[s, pl.ds(j * G, G)], gsem[s])

    def drain(dc, s_d):
      pltpu.make_async_copy(
          out_hbm.at[pl.ds(0, CH)], rows_v.at[s_d], gsem[s_d]).wait()
      pltpu.async_copy(
          rows_v.at[s_d], out_hbm.at[pl.ds(base + dc * CH, CH)], wsem[s_d])

    for s in range(NSLOT):
      fire(s, s)
      if s == NSLOT - 1:
        drain(0, 0)

    def body(t, carry):
      for s in range(NSLOT):
        c = t * NSLOT + s
        pltpu.make_async_copy(
            rows_v.at[s], out_hbm.at[pl.ds(0, CH)], wsem[s]).wait()
        fire(c, s)
        drain(c - (NSLOT - 1), (s + 1) % NSLOT)
      return carry

    lax.fori_loop(1, nch // NSLOT, body, 0)

    for q in range(NSLOT - 1):
      dc = nch - (NSLOT - 1) + q
      drain(dc, dc % NSLOT)
    for s in range(NSLOT):
      pltpu.make_async_copy(
          rows_v.at[s], out_hbm.at[pl.ds(0, CH)], wsem[s]).wait()

  return k


def kernel(indices, table):
  n = indices.size
  d = table.shape[1]
  out = _build(n, d)(indices.reshape(n), table)
  return out.reshape(*indices.shape, d)
